# baseline (speedup 1.0000x reference)
import functools

import jax
import jax.numpy as jnp
from jax import lax
from jax.experimental import pallas as pl
from jax.experimental.pallas import tpu as pltpu
from jax.experimental.pallas import tpu_sc as plsc

N = 10000
E = 160000
U = 80000
A = 160000
NR = 31
D = 64
NG = 16

BLK = 128
BP = 163840
NCORE = 2
NSUB = 16
NWORK = NCORE * NSUB


def _mesh():
    return plsc.VectorSubcoreMesh(
        core_axis_name="c", subcore_axis_name="s",
        num_cores=NCORE, num_subcores=NSUB)


@functools.lru_cache(maxsize=None)
def _make_gather(vs):
    n = len(vs)
    K = 4
    CH = K * BLK
    per_w = BP // NWORK
    blocks_w = per_w // BLK
    n_chunks = blocks_w // K
    n_pairs = n_chunks // 2

    def body(*refs):
        tabs = refs[:n]
        idxs = refs[n:2 * n]
        outs = refs[2 * n:3 * n]
        idx_v, r0, r1, gs0, gs1, os0, os1 = refs[3 * n:]
        cid = lax.axis_index("c")
        sid = lax.axis_index("s")
        wid = sid * NCORE + cid
        blk0 = wid * blocks_w

        def out_drain(buf, osem, s):
            pltpu.make_async_copy(
                buf, outs[s].at[pl.ds(0, CH), :], osem).wait()

        def gat_drain(buf, gsem, s):
            for _ in range(K):
                pltpu.make_async_copy(
                    tabs[s].at[pl.ds(0, BLK), :],
                    buf.at[pl.ds(0, BLK), :], gsem).wait()

        for s in range(n):
            pltpu.sync_copy(idxs[s].at[pl.ds(blk0, blocks_w), :], idx_v)

            def fire(i, buf, gsem, s=s):
                for j in range(K):
                    pltpu.async_copy(
                        tabs[s].at[idx_v.at[i * K + j]],
                        buf.at[pl.ds(j * BLK, BLK), :], gsem)

            def out_start(i, buf, osem, s=s):
                pltpu.async_copy(
                    buf, outs[s].at[pl.ds((blk0 + i * K) * BLK, CH), :],
                    osem)

            fire(0, r0, gs0)

            def pair(t, _, s=s):
                @pl.when(t > 0)
                def _():
                    out_drain(r1, os1, s)
                fire(2 * t + 1, r1, gs1)
                gat_drain(r0, gs0, s)
                out_start(2 * t, r0, os0)
                out_drain(r0, os0, s)

                @pl.when(t < n_pairs - 1)
                def _():
                    fire(2 * t + 2, r0, gs0)
                gat_drain(r1, gs1, s)
                out_start(2 * t + 1, r1, os1)
                return 0

            lax.fori_loop(0, n_pairs, pair, 0)
            out_drain(r1, os1, s)

    return pl.kernel(
        body,
        out_type=[jax.ShapeDtypeStruct((BP, D), jnp.float32)
                  for _ in range(n)],
        mesh=_mesh(),
        compiler_params=pltpu.CompilerParams(use_tc_tiling_on_sc=False),
        scratch_types=[
            pltpu.VMEM((blocks_w, BLK), jnp.int32),
            pltpu.VMEM((CH, D), jnp.float32),
            pltpu.VMEM((CH, D), jnp.float32),
            pltpu.SemaphoreType.DMA,
            pltpu.SemaphoreType.DMA,
            pltpu.SemaphoreType.DMA,
            pltpu.SemaphoreType.DMA,
        ],
    )


def _gather(*pairs):
    tables = tuple(t for t, _ in pairs)
    idxs = tuple(i for _, i in pairs)
    kern = _make_gather(tuple(t.shape[0] for t in tables))
    out = kern(*tables, *idxs)
    return out if isinstance(out, (tuple, list)) else (out,)


@functools.lru_cache(maxsize=None)
def _make_scatter(v_rows):
    K = 8
    CH = K * BLK
    per_t = BP // NSUB
    blocks_t = per_t // BLK
    n_chunks = blocks_t // K
    n_pairs = n_chunks // 2
    v16 = v_rows // NSUB

    def body(msg, idx2d, base, out, idx_v, m0, m1, shared, ms0, ms1, ssem):
        cid = lax.axis_index("c")
        sid = lax.axis_index("s")
        pltpu.sync_copy(idx2d.at[pl.ds(sid * blocks_t, blocks_t), :], idx_v)
        row0 = sid * per_t

        def msg_load(i, buf, msem, c0):
            pltpu.async_copy(
                msg.at[pl.ds(row0 + i * CH, CH), pl.ds(c0, 16)], buf, msem)

        def msg_drain(buf, msem, c0):
            pltpu.make_async_copy(
                msg.at[pl.ds(row0, CH), pl.ds(c0, 16)], buf, msem).wait()

        def adds_fire(i, buf):
            for j in range(K):
                pltpu.async_copy(
                    buf.at[pl.ds(j * BLK, BLK), :],
                    shared.at[idx_v.at[i * K + j]], ssem, add=True)

        def adds_drain(buf):
            for _ in range(K):
                pltpu.make_async_copy(
                    buf.at[pl.ds(0, BLK), :],
                    shared.at[pl.ds(0, BLK), :], ssem).wait()

        for g in range(2):
            c0 = cid * 32 + g * 16
            pltpu.sync_copy(
                base.at[pl.ds(sid * v16, v16), pl.ds(c0, 16)],
                shared.at[pl.ds(sid * v16, v16), :])
            plsc.subcore_barrier()
            msg_load(0, m0, ms0, c0)

            def pair(t, _, c0=c0):
                @pl.when(t > 0)
                def _():
                    adds_drain(m1)
                msg_load(2 * t + 1, m1, ms1, c0)
                msg_drain(m0, ms0, c0)
                adds_fire(2 * t, m0)
                adds_drain(m0)

                @pl.when(t < n_pairs - 1)
                def _():
                    msg_load(2 * t + 2, m0, ms0, c0)
                msg_drain(m1, ms1, c0)
                adds_fire(2 * t + 1, m1)
                return 0

            lax.fori_loop(0, n_pairs, pair, 0)
            adds_drain(m1)
            plsc.subcore_barrier()
            pltpu.sync_copy(
                shared.at[pl.ds(sid * v16, v16), :],
                out.at[pl.ds(sid * v16, v16), pl.ds(c0, 16)])
            plsc.subcore_barrier()

    return pl.kernel(
        body,
        out_type=jax.ShapeDtypeStruct((v_rows, D), jnp.float32),
        mesh=_mesh(),
        compiler_params=pltpu.CompilerParams(use_tc_tiling_on_sc=False),
        scratch_types=[
            pltpu.VMEM((blocks_t, BLK), jnp.int32),
            pltpu.VMEM((CH, 16), jnp.float32),
            pltpu.VMEM((CH, 16), jnp.float32),
            pltpu.VMEM_SHARED((v_rows, 16), jnp.float32),
            pltpu.SemaphoreType.DMA,
            pltpu.SemaphoreType.DMA,
            pltpu.SemaphoreType.DMA,
        ],
    )


def _scatter(msg, idx2d, base):
    return _make_scatter(base.shape[0])(msg, idx2d, base)


def _ln(x):
    m = jnp.mean(x, axis=-1, keepdims=True)
    v = jnp.mean((x - m) * (x - m), axis=-1, keepdims=True)
    return (x - m) * lax.rsqrt(v + 1e-5)


def _dot(a, b):
    return jnp.dot(a, b, preferred_element_type=jnp.float32)


_TC = pltpu.CompilerParams(dimension_semantics=("arbitrary",))
_BE = 2048


@functools.lru_cache(maxsize=None)
def _make_gated(n_in, n_valid):
    grid = (BP // _BE,)

    def body(*refs):
        ins = refs[:n_in]
        sc = refs[n_in]
        cw1, cb1, cw2, cb2, gw1, gb1, gw2, gb2, out = refs[n_in + 1:]
        w1 = cw1[...]
        v1 = gw1[...]
        cacc = cb1[...]
        gacc = gb1[...]
        for k in range(n_in):
            xk = ins[k][...]
            cacc = cacc + _dot(xk, w1[k * D:(k + 1) * D, :])
            gacc = gacc + _dot(xk, v1[k * D:(k + 1) * D, :])
        c = jax.nn.silu(cacc)
        c = jax.nn.silu(_ln(_dot(c, cw2[...]) + cb2[...]))
        g = jax.nn.silu(gacc)
        g = jax.nn.sigmoid(_ln(_dot(g, gw2[...]) + gb2[...]))
        val = c * g * sc[...]
        rid = pl.program_id(0) * _BE + lax.broadcasted_iota(
            jnp.int32, (_BE, 1), 0)
        out[...] = jnp.where(rid < n_valid, val, 0.0)

    def row_spec():
        return pl.BlockSpec((_BE, D), lambda i: (i, 0))

    def w_spec(shape):
        return pl.BlockSpec(shape, lambda i: tuple(0 for _ in shape))

    def call(ins, scale, cw1, cb1, cw2, cb2, gw1, gb1, gw2, gb2):
        in_specs = ([row_spec() for _ in range(n_in)] + [row_spec()]
                    + [w_spec(w.shape)
                       for w in (cw1, cb1, cw2, cb2, gw1, gb1, gw2, gb2)])
        return pl.pallas_call(
            body, grid=grid, in_specs=in_specs, out_specs=row_spec(),
            out_shape=jax.ShapeDtypeStruct((BP, D), jnp.float32),
            compiler_params=_TC,
        )(*ins, scale, cw1, cb1, cw2, cb2, gw1, gb1, gw2, gb2)

    return call


@functools.lru_cache(maxsize=None)
def _make_gated_lin(n_in):
    grid = (BP // _BE,)

    def body(*refs):
        ins = refs[:n_in]
        cw, cb, gw, gb, out = refs[n_in:]
        w = cw[...]
        v = gw[...]
        cacc = cb[...]
        gacc = gb[...]
        for k in range(n_in):
            xk = ins[k][...]
            cacc = cacc + _dot(xk, w[k * D:(k + 1) * D, :])
            gacc = gacc + _dot(xk, v[k * D:(k + 1) * D, :])
        out[...] = jax.nn.silu(_ln(cacc)) * jax.nn.sigmoid(_ln(gacc))

    def call(ins, cw, cb, gw, gb):
        row = pl.BlockSpec((_BE, D), lambda i: (i, 0))
        in_specs = ([row for _ in range(n_in)]
                    + [pl.BlockSpec(w.shape, lambda i: tuple(0 for _ in w.shape))
                       for w in (cw, cb, gw, gb)])
        return pl.pallas_call(
            body, grid=grid, in_specs=in_specs, out_specs=row,
            out_shape=jax.ShapeDtypeStruct((BP, D), jnp.float32),
            compiler_params=_TC,
        )(*ins, cw, cb, gw, gb)

    return call


def _pre_u(bb_ag, bb_bg, w_bond, w_ag, w_bg):
    bu = 2000
    grid = (U // bu,)

    def body(ar, br, wb, wa, wg, bond0, bwag, bwbg):
        a = ar[...]
        b = br[...]
        bond0[...] = _dot(a, wb[...])
        bwag[...] = _dot(a, wa[...])
        bwbg[...] = _dot(b, wg[...])

    row_in = pl.BlockSpec((bu, NR), lambda i: (i, 0))
    wsp = pl.BlockSpec((NR, D), lambda i: (0, 0))
    row_out = pl.BlockSpec((bu, D), lambda i: (i, 0))
    return pl.pallas_call(
        body, grid=grid,
        in_specs=[row_in, row_in, wsp, wsp, wsp],
        out_specs=[row_out, row_out, row_out],
        out_shape=[jax.ShapeDtypeStruct((U, D), jnp.float32)] * 3,
        compiler_params=_TC,
    )(bb_ag, bb_bg, w_bond, w_ag, w_bg)


def _pre_a(abp, w_angle):
    grid = (BP // _BE,)

    def body(ar, wr, out):
        out[...] = _dot(ar[...], wr[...])

    return pl.pallas_call(
        body, grid=grid,
        in_specs=[pl.BlockSpec((_BE, NR), lambda i: (i, 0)),
                  pl.BlockSpec((NR, D), lambda i: (0, 0))],
        out_specs=pl.BlockSpec((_BE, D), lambda i: (i, 0)),
        out_shape=jax.ShapeDtypeStruct((BP, D), jnp.float32),
        compiler_params=_TC,
    )(abp, w_angle)


def _x0(an2, emb):
    bn = 2000
    grid = (N // bn,)

    def body(ar, er, out):
        an = ar[...]
        oh = (an == lax.broadcasted_iota(jnp.int32, (bn, 94), 1))
        out[...] = _dot(oh.astype(jnp.float32), er[...])

    return pl.pallas_call(
        body, grid=grid,
        in_specs=[pl.BlockSpec((bn, 1), lambda i: (i, 0)),
                  pl.BlockSpec((94, D), lambda i: (0, 0))],
        out_specs=pl.BlockSpec((bn, D), lambda i: (i, 0)),
        out_shape=jax.ShapeDtypeStruct((N, D), jnp.float32),
        compiler_params=_TC,
    )(an2, emb)


def _readout(x, ow2, w1, b1, w2, b2, w3, b3, w4, b4):
    def body(xr, owr, w1r, b1r, w2r, b2r, w3r, b3r, w4r, b4r, out):
        h = _ln(xr[...])
        h = jax.nn.silu(_dot(h, w1r[...]) + b1r[...])
        h = jax.nn.silu(_dot(h, w2r[...]) + b2r[...])
        h = jax.nn.silu(_dot(h, w3r[...]) + b3r[...])
        e = _dot(h, w4r[...]) + b4r[...]
        oh = (owr[...] == lax.broadcasted_iota(jnp.int32, (N, NG), 1))
        oh = oh.astype(jnp.float32)
        dn = (((0,), (0,)), ((), ()))
        esum = lax.dot_general(e, oh, dn,
                               preferred_element_type=jnp.float32)
        cnt = lax.dot_general(jnp.ones_like(e), oh, dn,
                              preferred_element_type=jnp.float32)
        out[...] = esum / jnp.maximum(cnt, 1.0)

    return pl.pallas_call(
        body,
        out_shape=jax.ShapeDtypeStruct((1, NG), jnp.float32),
    )(x, ow2, w1, b1, w2, b2, w3, b3, w4, b4)


def _pad_idx(a):
    a = a.astype(jnp.int32)
    return jnp.pad(a, (0, BP - a.shape[0])).reshape(BP // BLK, BLK)


def kernel(atomic_numbers, atom_graph, directed2undirected, bg_center,
           bg_bond_i, bg_bond_j, atom_owners, bond_bases_ag, bond_bases_bg,
           angle_bases, params):
    p = params
    src = atom_graph[:, 0].astype(jnp.int32)
    perm_a = jnp.argsort(src)
    srcp = _pad_idx(src[perm_a])
    dstp = _pad_idx(atom_graph[:, 1].astype(jnp.int32)[perm_a])
    d2up = _pad_idx(directed2undirected.astype(jnp.int32)[perm_a])
    bgi = bg_bond_i.astype(jnp.int32)
    perm_b = jnp.argsort(bgi)
    bgcp = _pad_idx(bg_center.astype(jnp.int32)[perm_b])
    bgip = _pad_idx(bgi[perm_b])
    bgjp = _pad_idx(bg_bond_j.astype(jnp.int32)[perm_b])
    permbp = _pad_idx(perm_b)
    abp = jnp.pad(angle_bases, ((0, BP - A), (0, 0)))

    bond, bwag, bwbg = _pre_u(bond_bases_ag, bond_bases_bg,
                              p['bond_emb_w'], p['bw_ag_w'], p['bw_bg_w'])
    angle0 = _pre_a(abp, p['angle_emb_w'])
    x = _x0(atomic_numbers.reshape(N, 1).astype(jnp.int32), p['atom_emb'])
    bwd, bwg, angle = _gather((bwag, d2up), (bwbg, bgip), (angle0, permbp))

    gated3 = _make_gated(3, E)
    gated4 = _make_gated(4, A)
    glin4 = _make_gated_lin(4)

    def b2(v):
        return v.reshape(1, D)

    for i in range(4):
        center, nbr, bd = _gather((x, srcp), (x, dstp), (bond, d2up))
        msg = gated3([center, bd, nbr], bwd,
                     p['ac_cw1'][i], b2(p['ac_cb1'][i]),
                     p['ac_cw2'][i], b2(p['ac_cb2'][i]),
                     p['ac_gw1'][i], b2(p['ac_gb1'][i]),
                     p['ac_gw2'][i], b2(p['ac_gb2'][i]))
        x = _scatter(msg, srcp, x)
        if i < 3:
            ca, bi, bj = _gather((x, bgcp), (bond, bgip), (bond, bgjp))
            bmsg = gated4([bi, bj, angle, ca], bwg,
                          p['bc_cw1'][i], b2(p['bc_cb1'][i]),
                          p['bc_cw2'][i], b2(p['bc_cb2'][i]),
                          p['bc_gw1'][i], b2(p['bc_gb1'][i]),
                          p['bc_gw2'][i], b2(p['bc_gb2'][i]))
            bond = _scatter(bmsg, bgip, bond)
            bi2, bj2 = _gather((bond, bgip), (bond, bgjp))
            angle = glin4([bi2, bj2, angle, ca],
                          p['al_cw'][i], b2(p['al_cb'][i]),
                          p['al_gw'][i], b2(p['al_gb'][i]))

    out = _readout(x, atom_owners.reshape(N, 1).astype(jnp.int32),
                   p['h_w1'], b2(p['h_b1']), p['h_w2'], b2(p['h_b2']),
                   p['h_w3'], b2(p['h_b3']), p['h_w4'],
                   p['h_b4'].reshape(1, 1))
    return out.reshape(NG)


# speedup vs baseline: 1.0396x; 1.0396x over previous
import functools

import jax
import jax.numpy as jnp
from jax import lax
from jax.experimental import pallas as pl
from jax.experimental.pallas import tpu as pltpu
from jax.experimental.pallas import tpu_sc as plsc

N = 10000
E = 160000
U = 80000
A = 160000
NR = 31
D = 64
NG = 16

BLK = 128
BP = 163840
NCORE = 2
NSUB = 16
NWORK = NCORE * NSUB


def _mesh():
    return plsc.VectorSubcoreMesh(
        core_axis_name="c", subcore_axis_name="s",
        num_cores=NCORE, num_subcores=NSUB)


@functools.lru_cache(maxsize=None)
def _make_gather(vs):
    n = len(vs)
    K = 4
    CH = K * BLK
    per_w = BP // NWORK
    blocks_w = per_w // BLK
    n_chunks = blocks_w // K
    n_pairs = n_chunks // 2

    def body(*refs):
        tabs = refs[:n]
        idxs = refs[n:2 * n]
        outs = refs[2 * n:3 * n]
        idx_v, r0, r1, gs0, gs1, os0, os1 = refs[3 * n:]
        cid = lax.axis_index("c")
        sid = lax.axis_index("s")
        wid = sid * NCORE + cid
        blk0 = wid * blocks_w

        def out_drain(buf, osem, s):
            pltpu.make_async_copy(
                buf, outs[s].at[pl.ds(0, CH), :], osem).wait()

        def gat_drain(buf, gsem, s):
            for _ in range(K):
                pltpu.make_async_copy(
                    tabs[s].at[pl.ds(0, BLK), :],
                    buf.at[pl.ds(0, BLK), :], gsem).wait()

        for s in range(n):
            pltpu.sync_copy(idxs[s].at[pl.ds(blk0, blocks_w), :], idx_v)

            def fire(i, buf, gsem, s=s):
                for j in range(K):
                    pltpu.async_copy(
                        tabs[s].at[idx_v.at[i * K + j]],
                        buf.at[pl.ds(j * BLK, BLK), :], gsem)

            def out_start(i, buf, osem, s=s):
                pltpu.async_copy(
                    buf, outs[s].at[pl.ds((blk0 + i * K) * BLK, CH), :],
                    osem)

            fire(0, r0, gs0)

            def pair(t, _, s=s):
                @pl.when(t > 0)
                def _():
                    out_drain(r1, os1, s)
                fire(2 * t + 1, r1, gs1)
                gat_drain(r0, gs0, s)
                out_start(2 * t, r0, os0)
                out_drain(r0, os0, s)

                @pl.when(t < n_pairs - 1)
                def _():
                    fire(2 * t + 2, r0, gs0)
                gat_drain(r1, gs1, s)
                out_start(2 * t + 1, r1, os1)
                return 0

            lax.fori_loop(0, n_pairs, pair, 0)
            out_drain(r1, os1, s)

    return pl.kernel(
        body,
        out_type=[jax.ShapeDtypeStruct((BP, D), jnp.float32)
                  for _ in range(n)],
        mesh=_mesh(),
        compiler_params=pltpu.CompilerParams(use_tc_tiling_on_sc=False),
        scratch_types=[
            pltpu.VMEM((blocks_w, BLK), jnp.int32),
            pltpu.VMEM((CH, D), jnp.float32),
            pltpu.VMEM((CH, D), jnp.float32),
            pltpu.SemaphoreType.DMA,
            pltpu.SemaphoreType.DMA,
            pltpu.SemaphoreType.DMA,
            pltpu.SemaphoreType.DMA,
        ],
    )


def _gather(*pairs):
    tables = tuple(t for t, _ in pairs)
    idxs = tuple(i for _, i in pairs)
    kern = _make_gather(tuple(t.shape[0] for t in tables))
    out = kern(*tables, *idxs)
    return out if isinstance(out, (tuple, list)) else (out,)


@functools.lru_cache(maxsize=None)
def _make_scatter(v_rows):
    K = 8
    CH = K * BLK
    per_t = BP // NSUB
    blocks_t = per_t // BLK
    n_chunks = blocks_t // K
    n_pairs = n_chunks // 2
    v16 = v_rows // NSUB

    def body(msg, idx2d, base, out, idx_v, m0, m1, shared, ms0, ms1, ssem):
        cid = lax.axis_index("c")
        sid = lax.axis_index("s")
        pltpu.sync_copy(idx2d.at[pl.ds(sid * blocks_t, blocks_t), :], idx_v)
        row0 = sid * per_t

        def msg_load(i, buf, msem, c0):
            pltpu.async_copy(
                msg.at[pl.ds(row0 + i * CH, CH), pl.ds(c0, 16)], buf, msem)

        def msg_drain(buf, msem, c0):
            pltpu.make_async_copy(
                msg.at[pl.ds(row0, CH), pl.ds(c0, 16)], buf, msem).wait()

        def adds_fire(i, buf):
            for j in range(K):
                pltpu.async_copy(
                    buf.at[pl.ds(j * BLK, BLK), :],
                    shared.at[idx_v.at[i * K + j]], ssem, add=True)

        def adds_drain(buf):
            for _ in range(K):
                pltpu.make_async_copy(
                    buf.at[pl.ds(0, BLK), :],
                    shared.at[pl.ds(0, BLK), :], ssem).wait()

        for g in range(2):
            c0 = cid * 32 + g * 16
            pltpu.sync_copy(
                base.at[pl.ds(sid * v16, v16), pl.ds(c0, 16)],
                shared.at[pl.ds(sid * v16, v16), :])
            plsc.subcore_barrier()
            msg_load(0, m0, ms0, c0)

            def pair(t, _, c0=c0):
                @pl.when(t > 0)
                def _():
                    adds_drain(m1)
                msg_load(2 * t + 1, m1, ms1, c0)
                msg_drain(m0, ms0, c0)
                adds_fire(2 * t, m0)
                adds_drain(m0)

                @pl.when(t < n_pairs - 1)
                def _():
                    msg_load(2 * t + 2, m0, ms0, c0)
                msg_drain(m1, ms1, c0)
                adds_fire(2 * t + 1, m1)
                return 0

            lax.fori_loop(0, n_pairs, pair, 0)
            adds_drain(m1)
            plsc.subcore_barrier()
            pltpu.sync_copy(
                shared.at[pl.ds(sid * v16, v16), :],
                out.at[pl.ds(sid * v16, v16), pl.ds(c0, 16)])
            plsc.subcore_barrier()

    return pl.kernel(
        body,
        out_type=jax.ShapeDtypeStruct((v_rows, D), jnp.float32),
        mesh=_mesh(),
        compiler_params=pltpu.CompilerParams(use_tc_tiling_on_sc=False),
        scratch_types=[
            pltpu.VMEM((blocks_t, BLK), jnp.int32),
            pltpu.VMEM((CH, 16), jnp.float32),
            pltpu.VMEM((CH, 16), jnp.float32),
            pltpu.VMEM_SHARED((v_rows, 16), jnp.float32),
            pltpu.SemaphoreType.DMA,
            pltpu.SemaphoreType.DMA,
            pltpu.SemaphoreType.DMA,
        ],
    )


def _scatter(msg, idx2d, base):
    return _make_scatter(base.shape[0])(msg, idx2d, base)


def _ln(x):
    m = jnp.mean(x, axis=-1, keepdims=True)
    v = jnp.mean((x - m) * (x - m), axis=-1, keepdims=True)
    return (x - m) * lax.rsqrt(v + 1e-5)


def _dot(a, b):
    return jnp.dot(a, b, preferred_element_type=jnp.float32)


_TC = pltpu.CompilerParams(dimension_semantics=("arbitrary",))
_BE = 2048


@functools.lru_cache(maxsize=None)
def _make_gated(n_in, n_valid):
    grid = (BP // _BE,)

    def body(*refs):
        ins = refs[:n_in]
        sc = refs[n_in]
        cw1, cb1, cw2, cb2, gw1, gb1, gw2, gb2, out = refs[n_in + 1:]
        w1 = cw1[...]
        v1 = gw1[...]
        cacc = cb1[...]
        gacc = gb1[...]
        for k in range(n_in):
            xk = ins[k][...]
            cacc = cacc + _dot(xk, w1[k * D:(k + 1) * D, :])
            gacc = gacc + _dot(xk, v1[k * D:(k + 1) * D, :])
        c = jax.nn.silu(cacc)
        c = jax.nn.silu(_ln(_dot(c, cw2[...]) + cb2[...]))
        g = jax.nn.silu(gacc)
        g = jax.nn.sigmoid(_ln(_dot(g, gw2[...]) + gb2[...]))
        val = c * g * sc[...]
        rid = pl.program_id(0) * _BE + lax.broadcasted_iota(
            jnp.int32, (_BE, 1), 0)
        out[...] = jnp.where(rid < n_valid, val, 0.0)

    def row_spec():
        return pl.BlockSpec((_BE, D), lambda i: (i, 0))

    def w_spec(shape):
        return pl.BlockSpec(shape, lambda i: tuple(0 for _ in shape))

    def call(ins, scale, cw1, cb1, cw2, cb2, gw1, gb1, gw2, gb2):
        in_specs = ([row_spec() for _ in range(n_in)] + [row_spec()]
                    + [w_spec(w.shape)
                       for w in (cw1, cb1, cw2, cb2, gw1, gb1, gw2, gb2)])
        return pl.pallas_call(
            body, grid=grid, in_specs=in_specs, out_specs=row_spec(),
            out_shape=jax.ShapeDtypeStruct((BP, D), jnp.float32),
            compiler_params=_TC,
        )(*ins, scale, cw1, cb1, cw2, cb2, gw1, gb1, gw2, gb2)

    return call


@functools.lru_cache(maxsize=None)
def _make_gated_lin(n_in):
    grid = (BP // _BE,)

    def body(*refs):
        ins = refs[:n_in]
        cw, cb, gw, gb, out = refs[n_in:]
        w = cw[...]
        v = gw[...]
        cacc = cb[...]
        gacc = gb[...]
        for k in range(n_in):
            xk = ins[k][...]
            cacc = cacc + _dot(xk, w[k * D:(k + 1) * D, :])
            gacc = gacc + _dot(xk, v[k * D:(k + 1) * D, :])
        out[...] = jax.nn.silu(_ln(cacc)) * jax.nn.sigmoid(_ln(gacc))

    def call(ins, cw, cb, gw, gb):
        row = pl.BlockSpec((_BE, D), lambda i: (i, 0))
        in_specs = ([row for _ in range(n_in)]
                    + [pl.BlockSpec(w.shape, lambda i: tuple(0 for _ in w.shape))
                       for w in (cw, cb, gw, gb)])
        return pl.pallas_call(
            body, grid=grid, in_specs=in_specs, out_specs=row,
            out_shape=jax.ShapeDtypeStruct((BP, D), jnp.float32),
            compiler_params=_TC,
        )(*ins, cw, cb, gw, gb)

    return call


def _pre_u(bb_ag, bb_bg, w_bond, w_ag, w_bg):
    bu = 2000
    grid = (U // bu,)

    def body(ar, br, wb, wa, wg, bond0, bwag, bwbg):
        a = ar[...]
        b = br[...]
        bond0[...] = _dot(a, wb[...])
        bwag[...] = _dot(a, wa[...])
        bwbg[...] = _dot(b, wg[...])

    row_in = pl.BlockSpec((bu, NR), lambda i: (i, 0))
    wsp = pl.BlockSpec((NR, D), lambda i: (0, 0))
    row_out = pl.BlockSpec((bu, D), lambda i: (i, 0))
    return pl.pallas_call(
        body, grid=grid,
        in_specs=[row_in, row_in, wsp, wsp, wsp],
        out_specs=[row_out, row_out, row_out],
        out_shape=[jax.ShapeDtypeStruct((U, D), jnp.float32)] * 3,
        compiler_params=_TC,
    )(bb_ag, bb_bg, w_bond, w_ag, w_bg)


def _pre_a(abp, w_angle):
    grid = (BP // _BE,)

    def body(ar, wr, out):
        out[...] = _dot(ar[...], wr[...])

    return pl.pallas_call(
        body, grid=grid,
        in_specs=[pl.BlockSpec((_BE, NR), lambda i: (i, 0)),
                  pl.BlockSpec((NR, D), lambda i: (0, 0))],
        out_specs=pl.BlockSpec((_BE, D), lambda i: (i, 0)),
        out_shape=jax.ShapeDtypeStruct((BP, D), jnp.float32),
        compiler_params=_TC,
    )(abp, w_angle)


def _x0(an2, emb):
    bn = 2000
    grid = (N // bn,)

    def body(ar, er, out):
        an = ar[...]
        oh = (an == lax.broadcasted_iota(jnp.int32, (bn, 94), 1))
        out[...] = _dot(oh.astype(jnp.float32), er[...])

    return pl.pallas_call(
        body, grid=grid,
        in_specs=[pl.BlockSpec((bn, 1), lambda i: (i, 0)),
                  pl.BlockSpec((94, D), lambda i: (0, 0))],
        out_specs=pl.BlockSpec((bn, D), lambda i: (i, 0)),
        out_shape=jax.ShapeDtypeStruct((N, D), jnp.float32),
        compiler_params=_TC,
    )(an2, emb)


def _readout(x, ow2, w1, b1, w2, b2, w3, b3, w4, b4):
    def body(xr, owr, w1r, b1r, w2r, b2r, w3r, b3r, w4r, b4r, out):
        h = _ln(xr[...])
        h = jax.nn.silu(_dot(h, w1r[...]) + b1r[...])
        h = jax.nn.silu(_dot(h, w2r[...]) + b2r[...])
        h = jax.nn.silu(_dot(h, w3r[...]) + b3r[...])
        e = _dot(h, w4r[...]) + b4r[...]
        oh = (owr[...] == lax.broadcasted_iota(jnp.int32, (N, NG), 1))
        oh = oh.astype(jnp.float32)
        dn = (((0,), (0,)), ((), ()))
        esum = lax.dot_general(e, oh, dn,
                               preferred_element_type=jnp.float32)
        cnt = lax.dot_general(jnp.ones_like(e), oh, dn,
                              preferred_element_type=jnp.float32)
        out[...] = esum / jnp.maximum(cnt, 1.0)

    return pl.pallas_call(
        body,
        out_shape=jax.ShapeDtypeStruct((1, NG), jnp.float32),
    )(x, ow2, w1, b1, w2, b2, w3, b3, w4, b4)


def _pad_idx(a):
    a = a.astype(jnp.int32)
    return jnp.pad(a, (0, BP - a.shape[0])).reshape(BP // BLK, BLK)


def kernel(atomic_numbers, atom_graph, directed2undirected, bg_center,
           bg_bond_i, bg_bond_j, atom_owners, bond_bases_ag, bond_bases_bg,
           angle_bases, params):
    p = params
    srcp = _pad_idx(atom_graph[:, 0])
    dstp = _pad_idx(atom_graph[:, 1])
    d2up = _pad_idx(directed2undirected)
    bgcp = _pad_idx(bg_center)
    bgip = _pad_idx(bg_bond_i)
    bgjp = _pad_idx(bg_bond_j)
    abp = jnp.pad(angle_bases, ((0, BP - A), (0, 0)))

    bond, bwag, bwbg = _pre_u(bond_bases_ag, bond_bases_bg,
                              p['bond_emb_w'], p['bw_ag_w'], p['bw_bg_w'])
    angle = _pre_a(abp, p['angle_emb_w'])
    x = _x0(atomic_numbers.reshape(N, 1).astype(jnp.int32), p['atom_emb'])

    gated3 = _make_gated(3, E)
    gated4 = _make_gated(4, A)
    glin4 = _make_gated_lin(4)

    def b2(v):
        return v.reshape(1, D)

    bwd, bwg, center, nbr, bd = _gather(
        (bwag, d2up), (bwbg, bgip), (x, srcp), (x, dstp), (bond, d2up))
    for i in range(4):
        msg = gated3([center, bd, nbr], bwd,
                     p['ac_cw1'][i], b2(p['ac_cb1'][i]),
                     p['ac_cw2'][i], b2(p['ac_cb2'][i]),
                     p['ac_gw1'][i], b2(p['ac_gb1'][i]),
                     p['ac_gw2'][i], b2(p['ac_gb2'][i]))
        x = _scatter(msg, srcp, x)
        if i < 3:
            ca, bi, bj = _gather((x, bgcp), (bond, bgip), (bond, bgjp))
            bmsg = gated4([bi, bj, angle, ca], bwg,
                          p['bc_cw1'][i], b2(p['bc_cb1'][i]),
                          p['bc_cw2'][i], b2(p['bc_cb2'][i]),
                          p['bc_gw1'][i], b2(p['bc_gb1'][i]),
                          p['bc_gw2'][i], b2(p['bc_gb2'][i]))
            bond = _scatter(bmsg, bgip, bond)
            bi2, bj2, center, nbr, bd = _gather(
                (bond, bgip), (bond, bgjp),
                (x, srcp), (x, dstp), (bond, d2up))
            angle = glin4([bi2, bj2, angle, ca],
                          p['al_cw'][i], b2(p['al_cb'][i]),
                          p['al_gw'][i], b2(p['al_gb'][i]))

    out = _readout(x, atom_owners.reshape(N, 1).astype(jnp.int32),
                   p['h_w1'], b2(p['h_b1']), p['h_w2'], b2(p['h_b2']),
                   p['h_w3'], b2(p['h_b3']), p['h_w4'],
                   p['h_b4'].reshape(1, 1))
    return out.reshape(NG)


# speedup vs baseline: 1.0684x; 1.0277x over previous
import functools

import jax
import jax.numpy as jnp
from jax import lax
from jax.experimental import pallas as pl
from jax.experimental.pallas import tpu as pltpu
from jax.experimental.pallas import tpu_sc as plsc

N = 10000
E = 160000
U = 80000
A = 160000
NR = 31
D = 64
NG = 16

BLK = 128
BP = 163840
NCORE = 2
NSUB = 16
NWORK = NCORE * NSUB


def _mesh():
    return plsc.VectorSubcoreMesh(
        core_axis_name="c", subcore_axis_name="s",
        num_cores=NCORE, num_subcores=NSUB)


@functools.lru_cache(maxsize=None)
def _make_gather(vs):
    n = len(vs)
    K = 4
    CH = K * BLK
    per_w = BP // NWORK
    blocks_w = per_w // BLK
    n_chunks = blocks_w // K
    n_pairs = n_chunks // 2

    def body(*refs):
        tabs = refs[:n]
        idxs = refs[n:2 * n]
        outs = refs[2 * n:3 * n]
        idx_v, r0, r1, gs0, gs1, os0, os1 = refs[3 * n:]
        cid = lax.axis_index("c")
        sid = lax.axis_index("s")
        wid = sid * NCORE + cid
        blk0 = wid * blocks_w

        def out_drain(buf, osem, s):
            pltpu.make_async_copy(
                buf, outs[s].at[pl.ds(0, CH), :], osem).wait()

        def gat_drain(buf, gsem, s):
            for _ in range(K):
                pltpu.make_async_copy(
                    tabs[s].at[pl.ds(0, BLK), :],
                    buf.at[pl.ds(0, BLK), :], gsem).wait()

        for s in range(n):
            pltpu.sync_copy(idxs[s].at[pl.ds(blk0, blocks_w), :], idx_v)

            def fire(i, buf, gsem, s=s):
                for j in range(K):
                    pltpu.async_copy(
                        tabs[s].at[idx_v.at[i * K + j]],
                        buf.at[pl.ds(j * BLK, BLK), :], gsem)

            def out_start(i, buf, osem, s=s):
                pltpu.async_copy(
                    buf, outs[s].at[pl.ds((blk0 + i * K) * BLK, CH), :],
                    osem)

            fire(0, r0, gs0)

            def pair(t, _, s=s):
                @pl.when(t > 0)
                def _():
                    out_drain(r1, os1, s)
                fire(2 * t + 1, r1, gs1)
                gat_drain(r0, gs0, s)
                out_start(2 * t, r0, os0)
                out_drain(r0, os0, s)

                @pl.when(t < n_pairs - 1)
                def _():
                    fire(2 * t + 2, r0, gs0)
                gat_drain(r1, gs1, s)
                out_start(2 * t + 1, r1, os1)
                return 0

            lax.fori_loop(0, n_pairs, pair, 0)
            out_drain(r1, os1, s)

    return pl.kernel(
        body,
        out_type=[jax.ShapeDtypeStruct((BP, D), jnp.float32)
                  for _ in range(n)],
        mesh=_mesh(),
        compiler_params=pltpu.CompilerParams(use_tc_tiling_on_sc=False),
        scratch_types=[
            pltpu.VMEM((blocks_w, BLK), jnp.int32),
            pltpu.VMEM((CH, D), jnp.float32),
            pltpu.VMEM((CH, D), jnp.float32),
            pltpu.SemaphoreType.DMA,
            pltpu.SemaphoreType.DMA,
            pltpu.SemaphoreType.DMA,
            pltpu.SemaphoreType.DMA,
        ],
    )


def _gather(*pairs):
    tables = tuple(t for t, _ in pairs)
    idxs = tuple(i for _, i in pairs)
    kern = _make_gather(tuple(t.shape[0] for t in tables))
    out = kern(*tables, *idxs)
    return out if isinstance(out, (tuple, list)) else (out,)


@functools.lru_cache(maxsize=None)
def _make_scatter(v_rows):
    K = 8
    CH = K * BLK
    per_t = BP // NSUB
    blocks_t = per_t // BLK
    n_chunks = blocks_t // K
    n_pairs = n_chunks // 2
    v16 = v_rows // NSUB

    def body(msg, idx2d, base, out, idx_v, m0, m1, shared, ms0, ms1, ssem):
        cid = lax.axis_index("c")
        sid = lax.axis_index("s")
        pltpu.sync_copy(idx2d.at[pl.ds(sid * blocks_t, blocks_t), :], idx_v)
        row0 = sid * per_t

        def msg_load(i, buf, msem, c0):
            pltpu.async_copy(
                msg.at[pl.ds(row0 + i * CH, CH), pl.ds(c0, 16)], buf, msem)

        def msg_drain(buf, msem, c0):
            pltpu.make_async_copy(
                msg.at[pl.ds(row0, CH), pl.ds(c0, 16)], buf, msem).wait()

        def adds_fire(i, buf):
            for j in range(K):
                pltpu.async_copy(
                    buf.at[pl.ds(j * BLK, BLK), :],
                    shared.at[idx_v.at[i * K + j]], ssem, add=True)

        def adds_drain(buf):
            for _ in range(K):
                pltpu.make_async_copy(
                    buf.at[pl.ds(0, BLK), :],
                    shared.at[pl.ds(0, BLK), :], ssem).wait()

        for g in range(2):
            c0 = cid * 32 + g * 16
            pltpu.sync_copy(
                base.at[pl.ds(sid * v16, v16), pl.ds(c0, 16)],
                shared.at[pl.ds(sid * v16, v16), :])
            plsc.subcore_barrier()
            msg_load(0, m0, ms0, c0)

            def pair(t, _, c0=c0):
                @pl.when(t > 0)
                def _():
                    adds_drain(m1)
                msg_load(2 * t + 1, m1, ms1, c0)
                msg_drain(m0, ms0, c0)
                adds_fire(2 * t, m0)
                adds_drain(m0)

                @pl.when(t < n_pairs - 1)
                def _():
                    msg_load(2 * t + 2, m0, ms0, c0)
                msg_drain(m1, ms1, c0)
                adds_fire(2 * t + 1, m1)
                return 0

            lax.fori_loop(0, n_pairs, pair, 0)
            adds_drain(m1)
            plsc.subcore_barrier()
            pltpu.sync_copy(
                shared.at[pl.ds(sid * v16, v16), :],
                out.at[pl.ds(sid * v16, v16), pl.ds(c0, 16)])
            plsc.subcore_barrier()

    return pl.kernel(
        body,
        out_type=jax.ShapeDtypeStruct((v_rows, D), jnp.float32),
        mesh=_mesh(),
        compiler_params=pltpu.CompilerParams(use_tc_tiling_on_sc=False),
        scratch_types=[
            pltpu.VMEM((blocks_t, BLK), jnp.int32),
            pltpu.VMEM((CH, 16), jnp.float32),
            pltpu.VMEM((CH, 16), jnp.float32),
            pltpu.VMEM_SHARED((v_rows, 16), jnp.float32),
            pltpu.SemaphoreType.DMA,
            pltpu.SemaphoreType.DMA,
            pltpu.SemaphoreType.DMA,
        ],
    )


def _scatter(msg, idx2d, base):
    return _make_scatter(base.shape[0])(msg, idx2d, base)


def _ln(x):
    m = jnp.mean(x, axis=-1, keepdims=True)
    v = jnp.mean((x - m) * (x - m), axis=-1, keepdims=True)
    return (x - m) * lax.rsqrt(v + 1e-5)


def _dot(a, b):
    return jnp.dot(a, b, preferred_element_type=jnp.float32)


_TC = pltpu.CompilerParams(dimension_semantics=("arbitrary",))
_BE = 2048


@functools.lru_cache(maxsize=None)
def _make_gated(n_in, n_valid):
    grid = (BP // _BE,)

    def body(*refs):
        ins = refs[:n_in]
        sc = refs[n_in]
        cw1, cb1, cw2, cb2, gw1, gb1, gw2, gb2, out = refs[n_in + 1:]
        w1 = cw1[...]
        v1 = gw1[...]
        cacc = cb1[...]
        gacc = gb1[...]
        for k in range(n_in):
            xk = ins[k][...]
            cacc = cacc + _dot(xk, w1[k * D:(k + 1) * D, :])
            gacc = gacc + _dot(xk, v1[k * D:(k + 1) * D, :])
        c = jax.nn.silu(cacc)
        c = jax.nn.silu(_ln(_dot(c, cw2[...]) + cb2[...]))
        g = jax.nn.silu(gacc)
        g = jax.nn.sigmoid(_ln(_dot(g, gw2[...]) + gb2[...]))
        val = c * g * sc[...]
        rid = pl.program_id(0) * _BE + lax.broadcasted_iota(
            jnp.int32, (_BE, 1), 0)
        out[...] = jnp.where(rid < n_valid, val, 0.0)

    def row_spec():
        return pl.BlockSpec((_BE, D), lambda i: (i, 0))

    def w_spec(shape):
        return pl.BlockSpec(shape, lambda i: tuple(0 for _ in shape))

    def call(ins, scale, cw1, cb1, cw2, cb2, gw1, gb1, gw2, gb2):
        in_specs = ([row_spec() for _ in range(n_in)] + [row_spec()]
                    + [w_spec(w.shape)
                       for w in (cw1, cb1, cw2, cb2, gw1, gb1, gw2, gb2)])
        return pl.pallas_call(
            body, grid=grid, in_specs=in_specs, out_specs=row_spec(),
            out_shape=jax.ShapeDtypeStruct((BP, D), jnp.float32),
            compiler_params=_TC,
        )(*ins, scale, cw1, cb1, cw2, cb2, gw1, gb1, gw2, gb2)

    return call


@functools.lru_cache(maxsize=None)
def _make_gated_lin(n_in):
    grid = (BP // _BE,)

    def body(*refs):
        ins = refs[:n_in]
        cw, cb, gw, gb, out = refs[n_in:]
        w = cw[...]
        v = gw[...]
        cacc = cb[...]
        gacc = gb[...]
        for k in range(n_in):
            xk = ins[k][...]
            cacc = cacc + _dot(xk, w[k * D:(k + 1) * D, :])
            gacc = gacc + _dot(xk, v[k * D:(k + 1) * D, :])
        out[...] = jax.nn.silu(_ln(cacc)) * jax.nn.sigmoid(_ln(gacc))

    def call(ins, cw, cb, gw, gb):
        row = pl.BlockSpec((_BE, D), lambda i: (i, 0))
        in_specs = ([row for _ in range(n_in)]
                    + [pl.BlockSpec(w.shape, lambda i: tuple(0 for _ in w.shape))
                       for w in (cw, cb, gw, gb)])
        return pl.pallas_call(
            body, grid=grid, in_specs=in_specs, out_specs=row,
            out_shape=jax.ShapeDtypeStruct((BP, D), jnp.float32),
            compiler_params=_TC,
        )(*ins, cw, cb, gw, gb)

    return call


def _pre_u(bb_ag, bb_bg, w_bond, w_ag, w_bg):
    bu = 2000
    grid = (U // bu,)

    def body(ar, br, wb, wa, wg, bond0, bwag, bwbg):
        a = ar[...]
        b = br[...]
        bond0[...] = _dot(a, wb[...])
        bwag[...] = _dot(a, wa[...])
        bwbg[...] = _dot(b, wg[...])

    row_in = pl.BlockSpec((bu, NR), lambda i: (i, 0))
    wsp = pl.BlockSpec((NR, D), lambda i: (0, 0))
    row_out = pl.BlockSpec((bu, D), lambda i: (i, 0))
    return pl.pallas_call(
        body, grid=grid,
        in_specs=[row_in, row_in, wsp, wsp, wsp],
        out_specs=[row_out, row_out, row_out],
        out_shape=[jax.ShapeDtypeStruct((U, D), jnp.float32)] * 3,
        compiler_params=_TC,
    )(bb_ag, bb_bg, w_bond, w_ag, w_bg)


def _pre_a(abp, w_angle):
    grid = (BP // _BE,)

    def body(ar, wr, out):
        out[...] = _dot(ar[...], wr[...])

    return pl.pallas_call(
        body, grid=grid,
        in_specs=[pl.BlockSpec((_BE, NR), lambda i: (i, 0)),
                  pl.BlockSpec((NR, D), lambda i: (0, 0))],
        out_specs=pl.BlockSpec((_BE, D), lambda i: (i, 0)),
        out_shape=jax.ShapeDtypeStruct((BP, D), jnp.float32),
        compiler_params=_TC,
    )(abp, w_angle)


def _x0(an2, emb):
    bn = 2000
    grid = (N // bn,)

    def body(ar, er, out):
        an = ar[...]
        oh = (an == lax.broadcasted_iota(jnp.int32, (bn, 94), 1))
        out[...] = _dot(oh.astype(jnp.float32), er[...])

    return pl.pallas_call(
        body, grid=grid,
        in_specs=[pl.BlockSpec((bn, 1), lambda i: (i, 0)),
                  pl.BlockSpec((94, D), lambda i: (0, 0))],
        out_specs=pl.BlockSpec((bn, D), lambda i: (i, 0)),
        out_shape=jax.ShapeDtypeStruct((N, D), jnp.float32),
        compiler_params=_TC,
    )(an2, emb)


def _readout(x, ow2, w1, b1, w2, b2, w3, b3, w4, b4):
    def body(xr, owr, w1r, b1r, w2r, b2r, w3r, b3r, w4r, b4r, out):
        h = _ln(xr[...])
        h = jax.nn.silu(_dot(h, w1r[...]) + b1r[...])
        h = jax.nn.silu(_dot(h, w2r[...]) + b2r[...])
        h = jax.nn.silu(_dot(h, w3r[...]) + b3r[...])
        e = _dot(h, w4r[...]) + b4r[...]
        oh = (owr[...] == lax.broadcasted_iota(jnp.int32, (N, NG), 1))
        oh = oh.astype(jnp.float32)
        dn = (((0,), (0,)), ((), ()))
        esum = lax.dot_general(e, oh, dn,
                               preferred_element_type=jnp.float32)
        cnt = lax.dot_general(jnp.ones_like(e), oh, dn,
                              preferred_element_type=jnp.float32)
        out[...] = esum / jnp.maximum(cnt, 1.0)

    return pl.pallas_call(
        body,
        out_shape=jax.ShapeDtypeStruct((1, NG), jnp.float32),
    )(x, ow2, w1, b1, w2, b2, w3, b3, w4, b4)


def _pad_idx(a):
    a = a.astype(jnp.int32)
    return jnp.pad(a, (0, BP - a.shape[0])).reshape(BP // BLK, BLK)


def kernel(atomic_numbers, atom_graph, directed2undirected, bg_center,
           bg_bond_i, bg_bond_j, atom_owners, bond_bases_ag, bond_bases_bg,
           angle_bases, params):
    p = params
    srcp = _pad_idx(atom_graph[:, 0])
    dstp = _pad_idx(atom_graph[:, 1])
    d2up = _pad_idx(directed2undirected)
    bgcp = _pad_idx(bg_center)
    bgip = _pad_idx(bg_bond_i)
    bgjp = _pad_idx(bg_bond_j)
    abp = jnp.pad(angle_bases, ((0, BP - A), (0, 0)))

    bond, bwag, bwbg = _pre_u(bond_bases_ag, bond_bases_bg,
                              p['bond_emb_w'], p['bw_ag_w'], p['bw_bg_w'])
    angle = _pre_a(abp, p['angle_emb_w'])
    x = _x0(atomic_numbers.reshape(N, 1).astype(jnp.int32), p['atom_emb'])

    gated3 = _make_gated(3, E)
    gated4 = _make_gated(4, A)
    glin4 = _make_gated_lin(4)

    def b2(v):
        return v.reshape(1, D)

    bwd, bwg, center, nbr, bd = _gather(
        (bwag, d2up), (bwbg, bgip), (x, srcp), (x, dstp), (bond, d2up))
    for i in range(4):
        if i < 3:
            bi, bj = _gather((bond, bgip), (bond, bgjp))
        msg = gated3([center, bd, nbr], bwd,
                     p['ac_cw1'][i], b2(p['ac_cb1'][i]),
                     p['ac_cw2'][i], b2(p['ac_cb2'][i]),
                     p['ac_gw1'][i], b2(p['ac_gb1'][i]),
                     p['ac_gw2'][i], b2(p['ac_gb2'][i]))
        x = _scatter(msg, srcp, x)
        if i < 3:
            ca, = _gather((x, bgcp))
            center, nbr = _gather((x, srcp), (x, dstp))
            bmsg = gated4([bi, bj, angle, ca], bwg,
                          p['bc_cw1'][i], b2(p['bc_cb1'][i]),
                          p['bc_cw2'][i], b2(p['bc_cb2'][i]),
                          p['bc_gw1'][i], b2(p['bc_gb1'][i]),
                          p['bc_gw2'][i], b2(p['bc_gb2'][i]))
            bond = _scatter(bmsg, bgip, bond)
            bi2, bj2, bd = _gather((bond, bgip), (bond, bgjp),
                                   (bond, d2up))
            angle = glin4([bi2, bj2, angle, ca],
                          p['al_cw'][i], b2(p['al_cb'][i]),
                          p['al_gw'][i], b2(p['al_gb'][i]))

    out = _readout(x, atom_owners.reshape(N, 1).astype(jnp.int32),
                   p['h_w1'], b2(p['h_b1']), p['h_w2'], b2(p['h_b2']),
                   p['h_w3'], b2(p['h_b3']), p['h_w4'],
                   p['h_b4'].reshape(1, 1))
    return out.reshape(NG)


# speedup vs baseline: 1.1132x; 1.0419x over previous
import functools

import jax
import jax.numpy as jnp
from jax import lax
from jax.experimental import pallas as pl
from jax.experimental.pallas import tpu as pltpu
from jax.experimental.pallas import tpu_sc as plsc

N = 10000
E = 160000
U = 80000
A = 160000
NR = 31
D = 64
NG = 16

BLK = 128
BP = 163840
NCORE = 2
NSUB = 16
NWORK = NCORE * NSUB


def _mesh():
    return plsc.VectorSubcoreMesh(
        core_axis_name="c", subcore_axis_name="s",
        num_cores=NCORE, num_subcores=NSUB)


@functools.lru_cache(maxsize=None)
def _make_gather(vs):
    n = len(vs)
    K = 5
    CH = K * BLK
    per_w = BP // NWORK
    blocks_w = per_w // BLK
    n_chunks = blocks_w // K
    n_pairs = n_chunks // 2

    def body(*refs):
        tabs = refs[:n]
        idxs = refs[n:2 * n]
        outs = refs[2 * n:3 * n]
        idx_v, r0, r1, gs0, gs1, os0, os1 = refs[3 * n:]
        cid = lax.axis_index("c")
        sid = lax.axis_index("s")
        wid = sid * NCORE + cid
        blk0 = wid * blocks_w

        def out_drain(buf, osem, s):
            pltpu.make_async_copy(
                buf, outs[s].at[pl.ds(0, CH), :], osem).wait()

        def gat_drain(buf, gsem, s):
            for _ in range(K):
                pltpu.make_async_copy(
                    tabs[s].at[pl.ds(0, BLK), :],
                    buf.at[pl.ds(0, BLK), :], gsem).wait()

        for s in range(n):
            pltpu.sync_copy(idxs[s].at[pl.ds(blk0, blocks_w), :], idx_v)

            def fire(i, buf, gsem, s=s):
                for j in range(K):
                    pltpu.async_copy(
                        tabs[s].at[idx_v.at[i * K + j]],
                        buf.at[pl.ds(j * BLK, BLK), :], gsem)

            def out_start(i, buf, osem, s=s):
                pltpu.async_copy(
                    buf, outs[s].at[pl.ds((blk0 + i * K) * BLK, CH), :],
                    osem)

            fire(0, r0, gs0)

            def pair(t, _, s=s):
                @pl.when(t > 0)
                def _():
                    out_drain(r1, os1, s)
                fire(2 * t + 1, r1, gs1)
                gat_drain(r0, gs0, s)
                out_start(2 * t, r0, os0)
                out_drain(r0, os0, s)

                @pl.when(t < n_pairs - 1)
                def _():
                    fire(2 * t + 2, r0, gs0)
                gat_drain(r1, gs1, s)
                out_start(2 * t + 1, r1, os1)
                return 0

            lax.fori_loop(0, n_pairs, pair, 0)
            out_drain(r1, os1, s)

    return pl.kernel(
        body,
        out_type=[jax.ShapeDtypeStruct((BP, D), jnp.float32)
                  for _ in range(n)],
        mesh=_mesh(),
        compiler_params=pltpu.CompilerParams(use_tc_tiling_on_sc=False),
        scratch_types=[
            pltpu.VMEM((blocks_w, BLK), jnp.int32),
            pltpu.VMEM((CH, D), jnp.float32),
            pltpu.VMEM((CH, D), jnp.float32),
            pltpu.SemaphoreType.DMA,
            pltpu.SemaphoreType.DMA,
            pltpu.SemaphoreType.DMA,
            pltpu.SemaphoreType.DMA,
        ],
    )


def _gather(*pairs):
    tables = tuple(t for t, _ in pairs)
    idxs = tuple(i for _, i in pairs)
    kern = _make_gather(tuple(t.shape[0] for t in tables))
    out = kern(*tables, *idxs)
    return out if isinstance(out, (tuple, list)) else (out,)


@functools.lru_cache(maxsize=None)
def _make_scatter(v_rows):
    K = 8
    CH = K * BLK
    per_t = BP // NSUB
    blocks_t = per_t // BLK
    n_chunks = blocks_t // K
    n_pairs = n_chunks // 2
    v16 = v_rows // NSUB

    def body(msg, idx2d, base, out, idx_v, m0, m1, shared, ms0, ms1, ssem):
        cid = lax.axis_index("c")
        sid = lax.axis_index("s")
        pltpu.sync_copy(idx2d.at[pl.ds(sid * blocks_t, blocks_t), :], idx_v)
        row0 = sid * per_t

        def msg_load(i, buf, msem, c0):
            pltpu.async_copy(
                msg.at[pl.ds(row0 + i * CH, CH), pl.ds(c0, 16)], buf, msem)

        def msg_drain(buf, msem, c0):
            pltpu.make_async_copy(
                msg.at[pl.ds(row0, CH), pl.ds(c0, 16)], buf, msem).wait()

        def adds_fire(i, buf):
            for j in range(K):
                pltpu.async_copy(
                    buf.at[pl.ds(j * BLK, BLK), :],
                    shared.at[idx_v.at[i * K + j]], ssem, add=True)

        def adds_drain(buf):
            for _ in range(K):
                pltpu.make_async_copy(
                    buf.at[pl.ds(0, BLK), :],
                    shared.at[pl.ds(0, BLK), :], ssem).wait()

        for g in range(2):
            c0 = cid * 32 + g * 16
            pltpu.sync_copy(
                base.at[pl.ds(sid * v16, v16), pl.ds(c0, 16)],
                shared.at[pl.ds(sid * v16, v16), :])
            plsc.subcore_barrier()
            msg_load(0, m0, ms0, c0)

            def pair(t, _, c0=c0):
                @pl.when(t > 0)
                def _():
                    adds_drain(m1)
                msg_load(2 * t + 1, m1, ms1, c0)
                msg_drain(m0, ms0, c0)
                adds_fire(2 * t, m0)
                adds_drain(m0)

                @pl.when(t < n_pairs - 1)
                def _():
                    msg_load(2 * t + 2, m0, ms0, c0)
                msg_drain(m1, ms1, c0)
                adds_fire(2 * t + 1, m1)
                return 0

            lax.fori_loop(0, n_pairs, pair, 0)
            adds_drain(m1)
            plsc.subcore_barrier()
            pltpu.sync_copy(
                shared.at[pl.ds(sid * v16, v16), :],
                out.at[pl.ds(sid * v16, v16), pl.ds(c0, 16)])
            plsc.subcore_barrier()

    return pl.kernel(
        body,
        out_type=jax.ShapeDtypeStruct((v_rows, D), jnp.float32),
        mesh=_mesh(),
        compiler_params=pltpu.CompilerParams(use_tc_tiling_on_sc=False),
        scratch_types=[
            pltpu.VMEM((blocks_t, BLK), jnp.int32),
            pltpu.VMEM((CH, 16), jnp.float32),
            pltpu.VMEM((CH, 16), jnp.float32),
            pltpu.VMEM_SHARED((v_rows, 16), jnp.float32),
            pltpu.SemaphoreType.DMA,
            pltpu.SemaphoreType.DMA,
            pltpu.SemaphoreType.DMA,
        ],
    )


def _scatter(msg, idx2d, base):
    return _make_scatter(base.shape[0])(msg, idx2d, base)


def _ln(x):
    m = jnp.mean(x, axis=-1, keepdims=True)
    v = jnp.mean((x - m) * (x - m), axis=-1, keepdims=True)
    return (x - m) * lax.rsqrt(v + 1e-5)


def _dot(a, b):
    return jnp.dot(a, b, preferred_element_type=jnp.float32)


_TC = pltpu.CompilerParams(dimension_semantics=("arbitrary",))
_BE = 2048


@functools.lru_cache(maxsize=None)
def _make_gated(n_in, n_valid):
    grid = (BP // _BE,)

    def body(*refs):
        ins = refs[:n_in]
        sc = refs[n_in]
        cw1, cb1, cw2, cb2, gw1, gb1, gw2, gb2, out = refs[n_in + 1:]
        w1 = cw1[...]
        v1 = gw1[...]
        cacc = cb1[...]
        gacc = gb1[...]
        for k in range(n_in):
            xk = ins[k][...]
            cacc = cacc + _dot(xk, w1[k * D:(k + 1) * D, :])
            gacc = gacc + _dot(xk, v1[k * D:(k + 1) * D, :])
        c = jax.nn.silu(cacc)
        c = jax.nn.silu(_ln(_dot(c, cw2[...]) + cb2[...]))
        g = jax.nn.silu(gacc)
        g = jax.nn.sigmoid(_ln(_dot(g, gw2[...]) + gb2[...]))
        val = c * g * sc[...]
        rid = pl.program_id(0) * _BE + lax.broadcasted_iota(
            jnp.int32, (_BE, 1), 0)
        out[...] = jnp.where(rid < n_valid, val, 0.0)

    def row_spec():
        return pl.BlockSpec((_BE, D), lambda i: (i, 0))

    def w_spec(shape):
        return pl.BlockSpec(shape, lambda i: tuple(0 for _ in shape))

    def call(ins, scale, cw1, cb1, cw2, cb2, gw1, gb1, gw2, gb2):
        in_specs = ([row_spec() for _ in range(n_in)] + [row_spec()]
                    + [w_spec(w.shape)
                       for w in (cw1, cb1, cw2, cb2, gw1, gb1, gw2, gb2)])
        return pl.pallas_call(
            body, grid=grid, in_specs=in_specs, out_specs=row_spec(),
            out_shape=jax.ShapeDtypeStruct((BP, D), jnp.float32),
            compiler_params=_TC,
        )(*ins, scale, cw1, cb1, cw2, cb2, gw1, gb1, gw2, gb2)

    return call


@functools.lru_cache(maxsize=None)
def _make_gated_lin(n_in):
    grid = (BP // _BE,)

    def body(*refs):
        ins = refs[:n_in]
        cw, cb, gw, gb, out = refs[n_in:]
        w = cw[...]
        v = gw[...]
        cacc = cb[...]
        gacc = gb[...]
        for k in range(n_in):
            xk = ins[k][...]
            cacc = cacc + _dot(xk, w[k * D:(k + 1) * D, :])
            gacc = gacc + _dot(xk, v[k * D:(k + 1) * D, :])
        out[...] = jax.nn.silu(_ln(cacc)) * jax.nn.sigmoid(_ln(gacc))

    def call(ins, cw, cb, gw, gb):
        row = pl.BlockSpec((_BE, D), lambda i: (i, 0))
        in_specs = ([row for _ in range(n_in)]
                    + [pl.BlockSpec(w.shape, lambda i: tuple(0 for _ in w.shape))
                       for w in (cw, cb, gw, gb)])
        return pl.pallas_call(
            body, grid=grid, in_specs=in_specs, out_specs=row,
            out_shape=jax.ShapeDtypeStruct((BP, D), jnp.float32),
            compiler_params=_TC,
        )(*ins, cw, cb, gw, gb)

    return call


def _pre_u(bb_ag, bb_bg, w_bond, w_ag, w_bg):
    bu = 2000
    grid = (U // bu,)

    def body(ar, br, wb, wa, wg, bond0, bwag, bwbg):
        a = ar[...]
        b = br[...]
        bond0[...] = _dot(a, wb[...])
        bwag[...] = _dot(a, wa[...])
        bwbg[...] = _dot(b, wg[...])

    row_in = pl.BlockSpec((bu, NR), lambda i: (i, 0))
    wsp = pl.BlockSpec((NR, D), lambda i: (0, 0))
    row_out = pl.BlockSpec((bu, D), lambda i: (i, 0))
    return pl.pallas_call(
        body, grid=grid,
        in_specs=[row_in, row_in, wsp, wsp, wsp],
        out_specs=[row_out, row_out, row_out],
        out_shape=[jax.ShapeDtypeStruct((U, D), jnp.float32)] * 3,
        compiler_params=_TC,
    )(bb_ag, bb_bg, w_bond, w_ag, w_bg)


def _pre_a(abp, w_angle):
    grid = (BP // _BE,)

    def body(ar, wr, out):
        out[...] = _dot(ar[...], wr[...])

    return pl.pallas_call(
        body, grid=grid,
        in_specs=[pl.BlockSpec((_BE, NR), lambda i: (i, 0)),
                  pl.BlockSpec((NR, D), lambda i: (0, 0))],
        out_specs=pl.BlockSpec((_BE, D), lambda i: (i, 0)),
        out_shape=jax.ShapeDtypeStruct((BP, D), jnp.float32),
        compiler_params=_TC,
    )(abp, w_angle)


def _x0(an2, emb):
    bn = 2000
    grid = (N // bn,)

    def body(ar, er, out):
        an = ar[...]
        oh = (an == lax.broadcasted_iota(jnp.int32, (bn, 94), 1))
        out[...] = _dot(oh.astype(jnp.float32), er[...])

    return pl.pallas_call(
        body, grid=grid,
        in_specs=[pl.BlockSpec((bn, 1), lambda i: (i, 0)),
                  pl.BlockSpec((94, D), lambda i: (0, 0))],
        out_specs=pl.BlockSpec((bn, D), lambda i: (i, 0)),
        out_shape=jax.ShapeDtypeStruct((N, D), jnp.float32),
        compiler_params=_TC,
    )(an2, emb)


def _readout(x, ow2, w1, b1, w2, b2, w3, b3, w4, b4):
    def body(xr, owr, w1r, b1r, w2r, b2r, w3r, b3r, w4r, b4r, out):
        h = _ln(xr[...])
        h = jax.nn.silu(_dot(h, w1r[...]) + b1r[...])
        h = jax.nn.silu(_dot(h, w2r[...]) + b2r[...])
        h = jax.nn.silu(_dot(h, w3r[...]) + b3r[...])
        e = _dot(h, w4r[...]) + b4r[...]
        oh = (owr[...] == lax.broadcasted_iota(jnp.int32, (N, NG), 1))
        oh = oh.astype(jnp.float32)
        dn = (((0,), (0,)), ((), ()))
        esum = lax.dot_general(e, oh, dn,
                               preferred_element_type=jnp.float32)
        cnt = lax.dot_general(jnp.ones_like(e), oh, dn,
                              preferred_element_type=jnp.float32)
        out[...] = esum / jnp.maximum(cnt, 1.0)

    return pl.pallas_call(
        body,
        out_shape=jax.ShapeDtypeStruct((1, NG), jnp.float32),
    )(x, ow2, w1, b1, w2, b2, w3, b3, w4, b4)


def _pad_idx(a):
    a = a.astype(jnp.int32)
    return jnp.pad(a, (0, BP - a.shape[0])).reshape(BP // BLK, BLK)


def kernel(atomic_numbers, atom_graph, directed2undirected, bg_center,
           bg_bond_i, bg_bond_j, atom_owners, bond_bases_ag, bond_bases_bg,
           angle_bases, params):
    p = params
    srcp = _pad_idx(atom_graph[:, 0])
    dstp = _pad_idx(atom_graph[:, 1])
    d2up = _pad_idx(directed2undirected)
    bgcp = _pad_idx(bg_center)
    bgip = _pad_idx(bg_bond_i)
    bgjp = _pad_idx(bg_bond_j)
    abp = jnp.pad(angle_bases, ((0, BP - A), (0, 0)))

    bond, bwag, bwbg = _pre_u(bond_bases_ag, bond_bases_bg,
                              p['bond_emb_w'], p['bw_ag_w'], p['bw_bg_w'])
    angle = _pre_a(abp, p['angle_emb_w'])
    x = _x0(atomic_numbers.reshape(N, 1).astype(jnp.int32), p['atom_emb'])

    gated3 = _make_gated(3, E)
    gated4 = _make_gated(4, A)
    glin4 = _make_gated_lin(4)

    def b2(v):
        return v.reshape(1, D)

    bwd, bwg = _gather((bwag, d2up), (bwbg, bgip))
    for i in range(4):
        center, nbr, bd = _gather((x, srcp), (x, dstp), (bond, d2up))
        msg = gated3([center, bd, nbr], bwd,
                     p['ac_cw1'][i], b2(p['ac_cb1'][i]),
                     p['ac_cw2'][i], b2(p['ac_cb2'][i]),
                     p['ac_gw1'][i], b2(p['ac_gb1'][i]),
                     p['ac_gw2'][i], b2(p['ac_gb2'][i]))
        x = _scatter(msg, srcp, x)
        if i < 3:
            ca, bi, bj = _gather((x, bgcp), (bond, bgip), (bond, bgjp))
            bmsg = gated4([bi, bj, angle, ca], bwg,
                          p['bc_cw1'][i], b2(p['bc_cb1'][i]),
                          p['bc_cw2'][i], b2(p['bc_cb2'][i]),
                          p['bc_gw1'][i], b2(p['bc_gb1'][i]),
                          p['bc_gw2'][i], b2(p['bc_gb2'][i]))
            bond = _scatter(bmsg, bgip, bond)
            bi2, bj2 = _gather((bond, bgip), (bond, bgjp))
            angle = glin4([bi2, bj2, angle, ca],
                          p['al_cw'][i], b2(p['al_cb'][i]),
                          p['al_gw'][i], b2(p['al_gb'][i]))

    out = _readout(x, atom_owners.reshape(N, 1).astype(jnp.int32),
                   p['h_w1'], b2(p['h_b1']), p['h_w2'], b2(p['h_b2']),
                   p['h_w3'], b2(p['h_b3']), p['h_w4'],
                   p['h_b4'].reshape(1, 1))
    return out.reshape(NG)


# speedup vs baseline: 1.3837x; 1.2430x over previous
import functools

import jax
import jax.numpy as jnp
from jax import lax
from jax.experimental import pallas as pl
from jax.experimental.pallas import tpu as pltpu
from jax.experimental.pallas import tpu_sc as plsc

N = 10000
E = 160000
U = 80000
A = 160000
NR = 31
D = 64
NG = 16

BLK = 128
BP = 163840
BH = BP // 2
NCORE = 2
NSUB = 16
NWORK = NCORE * NSUB


def _mesh():
    return plsc.VectorSubcoreMesh(
        core_axis_name="c", subcore_axis_name="s",
        num_cores=NCORE, num_subcores=NSUB)


@functools.lru_cache(maxsize=None)
def _make_gather(vs):
    n = len(vs)
    K = 5
    CH = K * BLK
    per_w = BP // NWORK
    blocks_w = per_w // BLK
    n_chunks = blocks_w // K
    n_pairs = n_chunks // 2

    def body(*refs):
        tabs = refs[:n]
        idxs = refs[n:2 * n]
        outs = refs[2 * n:3 * n]
        idx_v, r0, r1, gs0, gs1, os0, os1 = refs[3 * n:]
        cid = lax.axis_index("c")
        sid = lax.axis_index("s")
        wid = sid * NCORE + cid
        blk0 = wid * blocks_w
        half = wid // NSUB
        c_off = half * D
        prow0 = blk0 * BLK - half * BH

        def out_drain(buf, osem, s):
            pltpu.make_async_copy(
                buf, outs[s].at[pl.ds(0, CH), pl.ds(0, D)], osem).wait()

        def gat_drain(buf, gsem, s):
            for _ in range(K):
                pltpu.make_async_copy(
                    tabs[s].at[pl.ds(0, BLK), :],
                    buf.at[pl.ds(0, BLK), :], gsem).wait()

        for s in range(n):
            pltpu.sync_copy(idxs[s].at[pl.ds(blk0, blocks_w), :], idx_v)

            def fire(i, buf, gsem, s=s):
                for j in range(K):
                    pltpu.async_copy(
                        tabs[s].at[idx_v.at[i * K + j]],
                        buf.at[pl.ds(j * BLK, BLK), :], gsem)

            def out_start(i, buf, osem, s=s):
                pltpu.async_copy(
                    buf,
                    outs[s].at[pl.ds(prow0 + i * CH, CH), pl.ds(c_off, D)],
                    osem)

            fire(0, r0, gs0)

            def pair(t, _, s=s):
                @pl.when(t > 0)
                def _():
                    out_drain(r1, os1, s)
                fire(2 * t + 1, r1, gs1)
                gat_drain(r0, gs0, s)
                out_start(2 * t, r0, os0)
                out_drain(r0, os0, s)

                @pl.when(t < n_pairs - 1)
                def _():
                    fire(2 * t + 2, r0, gs0)
                gat_drain(r1, gs1, s)
                out_start(2 * t + 1, r1, os1)
                return 0

            lax.fori_loop(0, n_pairs, pair, 0)
            out_drain(r1, os1, s)

    return pl.kernel(
        body,
        out_type=[jax.ShapeDtypeStruct((BH, 2 * D), jnp.float32)
                  for _ in range(n)],
        mesh=_mesh(),
        compiler_params=pltpu.CompilerParams(use_tc_tiling_on_sc=False),
        scratch_types=[
            pltpu.VMEM((blocks_w, BLK), jnp.int32),
            pltpu.VMEM((CH, D), jnp.float32),
            pltpu.VMEM((CH, D), jnp.float32),
            pltpu.SemaphoreType.DMA,
            pltpu.SemaphoreType.DMA,
            pltpu.SemaphoreType.DMA,
            pltpu.SemaphoreType.DMA,
        ],
    )


def _gather(*pairs):
    tables = tuple(t for t, _ in pairs)
    idxs = tuple(i for _, i in pairs)
    kern = _make_gather(tuple(t.shape[0] for t in tables))
    out = kern(*tables, *idxs)
    return out if isinstance(out, (tuple, list)) else (out,)


@functools.lru_cache(maxsize=None)
def _make_scatter(v_rows):
    K = 8
    CH = K * BLK
    per_t = BP // NSUB
    blocks_t = per_t // BLK
    n_chunks = blocks_t // K
    n_pairs = n_chunks // 2
    v16 = v_rows // NSUB

    def body(msg, idx2d, base, out, idx_v, m0, m1, shared, ms0, ms1, ssem):
        cid = lax.axis_index("c")
        sid = lax.axis_index("s")
        pltpu.sync_copy(idx2d.at[pl.ds(sid * blocks_t, blocks_t), :], idx_v)
        half = sid // (NSUB // 2)
        c_base = half * D
        prow0 = sid * per_t - half * BH

        def msg_load(i, buf, msem, c0):
            pltpu.async_copy(
                msg.at[pl.ds(prow0 + i * CH, CH), pl.ds(c_base + c0, 16)],
                buf, msem)

        def msg_drain(buf, msem, c0):
            pltpu.make_async_copy(
                msg.at[pl.ds(0, CH), pl.ds(0, 16)], buf, msem).wait()

        def adds_fire(i, buf):
            for j in range(K):
                pltpu.async_copy(
                    buf.at[pl.ds(j * BLK, BLK), :],
                    shared.at[idx_v.at[i * K + j]], ssem, add=True)

        def adds_drain(buf):
            for _ in range(K):
                pltpu.make_async_copy(
                    buf.at[pl.ds(0, BLK), :],
                    shared.at[pl.ds(0, BLK), :], ssem).wait()

        for g in range(2):
            c0 = cid * 32 + g * 16
            pltpu.sync_copy(
                base.at[pl.ds(sid * v16, v16), pl.ds(c0, 16)],
                shared.at[pl.ds(sid * v16, v16), :])
            plsc.subcore_barrier()
            msg_load(0, m0, ms0, c0)

            def pair(t, _, c0=c0):
                @pl.when(t > 0)
                def _():
                    adds_drain(m1)
                msg_load(2 * t + 1, m1, ms1, c0)
                msg_drain(m0, ms0, c0)
                adds_fire(2 * t, m0)
                adds_drain(m0)

                @pl.when(t < n_pairs - 1)
                def _():
                    msg_load(2 * t + 2, m0, ms0, c0)
                msg_drain(m1, ms1, c0)
                adds_fire(2 * t + 1, m1)
                return 0

            lax.fori_loop(0, n_pairs, pair, 0)
            adds_drain(m1)
            plsc.subcore_barrier()
            pltpu.sync_copy(
                shared.at[pl.ds(sid * v16, v16), :],
                out.at[pl.ds(sid * v16, v16), pl.ds(c0, 16)])
            plsc.subcore_barrier()

    return pl.kernel(
        body,
        out_type=jax.ShapeDtypeStruct((v_rows, D), jnp.float32),
        mesh=_mesh(),
        compiler_params=pltpu.CompilerParams(use_tc_tiling_on_sc=False),
        scratch_types=[
            pltpu.VMEM((blocks_t, BLK), jnp.int32),
            pltpu.VMEM((CH, 16), jnp.float32),
            pltpu.VMEM((CH, 16), jnp.float32),
            pltpu.VMEM_SHARED((v_rows, 16), jnp.float32),
            pltpu.SemaphoreType.DMA,
            pltpu.SemaphoreType.DMA,
            pltpu.SemaphoreType.DMA,
        ],
    )


def _scatter(msg, idx2d, base):
    return _make_scatter(base.shape[0])(msg, idx2d, base)


def _ln(x):
    m = jnp.mean(x, axis=-1, keepdims=True)
    v = jnp.mean((x - m) * (x - m), axis=-1, keepdims=True)
    return (x - m) * lax.rsqrt(v + 1e-5)


def _dot(a, b):
    return jnp.dot(a, b, preferred_element_type=jnp.float32)


_TC = pltpu.CompilerParams(dimension_semantics=("arbitrary",))
_BE = 2048
_BE2 = 1024


@functools.lru_cache(maxsize=None)
def _make_gated(kinds, n_valid, two_layer, scaled, out_halves):
    n_slot = len(kinds)
    n_in = sum(2 if k == 'h' else 1 for k in kinds)
    n_w = 8 if two_layer else 4
    grid = (BH // _BE2,)

    def body(*refs):
        it = iter(refs)
        ins = []
        for k in kinds:
            if k == 'p':
                ins.append(('p', next(it)))
            else:
                ins.append(('h', (next(it), next(it))))
        sc = next(it) if scaled else None
        ws = [next(it) for _ in range(n_w)]
        outs = list(it)
        if two_layer:
            cw1, cb1, cw2, cb2, gw1, gb1, gw2, gb2 = ws
        else:
            cw1, cb1, gw1, gb1 = ws
        w1 = cw1[...]
        v1 = gw1[...]
        scv = sc[...] if scaled else None
        halves = []
        for h in range(2):
            cacc = cb1[...]
            gacc = gb1[...]
            for k, (kind, r) in enumerate(ins):
                if kind == 'p':
                    xk = r[...][:, h * D:(h + 1) * D]
                else:
                    xk = r[h][...]
                cacc = cacc + _dot(xk, w1[k * D:(k + 1) * D, :])
                gacc = gacc + _dot(xk, v1[k * D:(k + 1) * D, :])
            if two_layer:
                c = jax.nn.silu(cacc)
                c = jax.nn.silu(_ln(_dot(c, cw2[...]) + cb2[...]))
                g = jax.nn.silu(gacc)
                g = jax.nn.sigmoid(_ln(_dot(g, gw2[...]) + gb2[...]))
            else:
                c = jax.nn.silu(_ln(cacc))
                g = jax.nn.sigmoid(_ln(gacc))
            val = c * g
            if scaled:
                val = val * scv[:, h * D:(h + 1) * D]
                if h == 1:
                    rid = pl.program_id(0) * _BE2 + lax.broadcasted_iota(
                        jnp.int32, (_BE2, 1), 0)
                    val = jnp.where(rid < n_valid - BH, val, 0.0)
            halves.append(val)
        if out_halves:
            outs[0][...] = halves[0]
            outs[1][...] = halves[1]
        else:
            outs[0][...] = jnp.concatenate(halves, axis=1)

    pair = pl.BlockSpec((_BE2, 2 * D), lambda i: (i, 0))
    halfs = pl.BlockSpec((_BE2, D), lambda i: (i, 0))

    def w_spec(shape):
        return pl.BlockSpec(shape, lambda i: tuple(0 for _ in shape))

    def call(ins, scale, *weights):
        flat = []
        in_specs = []
        for kind, v in zip(kinds, ins):
            if kind == 'p':
                flat.append(v)
                in_specs.append(pair)
            else:
                flat.extend(v)
                in_specs.extend([halfs, halfs])
        if scaled:
            flat.append(scale)
            in_specs.append(pair)
        flat.extend(weights)
        in_specs.extend(w_spec(w.shape) for w in weights)
        if out_halves:
            out_specs = [halfs, halfs]
            out_shape = [jax.ShapeDtypeStruct((BH, D), jnp.float32)] * 2
        else:
            out_specs = pair
            out_shape = jax.ShapeDtypeStruct((BH, 2 * D), jnp.float32)
        return pl.pallas_call(
            body, grid=grid, in_specs=in_specs, out_specs=out_specs,
            out_shape=out_shape, compiler_params=_TC,
        )(*flat)

    return call


def _pre_u(bb_ag, bb_bg, w_bond, w_ag, w_bg):
    bu = 2000
    grid = (U // bu,)

    def body(ar, br, wb, wa, wg, bond0, bwag, bwbg):
        a = ar[...]
        b = br[...]
        bond0[...] = _dot(a, wb[...])
        bwag[...] = _dot(a, wa[...])
        bwbg[...] = _dot(b, wg[...])

    row_in = pl.BlockSpec((bu, NR), lambda i: (i, 0))
    wsp = pl.BlockSpec((NR, D), lambda i: (0, 0))
    row_out = pl.BlockSpec((bu, D), lambda i: (i, 0))
    return pl.pallas_call(
        body, grid=grid,
        in_specs=[row_in, row_in, wsp, wsp, wsp],
        out_specs=[row_out, row_out, row_out],
        out_shape=[jax.ShapeDtypeStruct((U, D), jnp.float32)] * 3,
        compiler_params=_TC,
    )(bb_ag, bb_bg, w_bond, w_ag, w_bg)


def _pre_a(abh, w_angle):
    grid = (BH // _BE,)

    def body(ar, wr, out):
        out[...] = _dot(ar[...], wr[...])

    return pl.pallas_call(
        body, grid=grid,
        in_specs=[pl.BlockSpec((_BE, NR), lambda i: (i, 0)),
                  pl.BlockSpec((NR, D), lambda i: (0, 0))],
        out_specs=pl.BlockSpec((_BE, D), lambda i: (i, 0)),
        out_shape=jax.ShapeDtypeStruct((BH, D), jnp.float32),
        compiler_params=_TC,
    )(abh, w_angle)


def _x0(an2, emb):
    bn = 2000
    grid = (N // bn,)

    def body(ar, er, out):
        an = ar[...]
        oh = (an == lax.broadcasted_iota(jnp.int32, (bn, 94), 1))
        out[...] = _dot(oh.astype(jnp.float32), er[...])

    return pl.pallas_call(
        body, grid=grid,
        in_specs=[pl.BlockSpec((bn, 1), lambda i: (i, 0)),
                  pl.BlockSpec((94, D), lambda i: (0, 0))],
        out_specs=pl.BlockSpec((bn, D), lambda i: (i, 0)),
        out_shape=jax.ShapeDtypeStruct((N, D), jnp.float32),
        compiler_params=_TC,
    )(an2, emb)


def _readout(x, ow2, w1, b1, w2, b2, w3, b3, w4, b4):
    def body(xr, owr, w1r, b1r, w2r, b2r, w3r, b3r, w4r, b4r, out):
        h = _ln(xr[...])
        h = jax.nn.silu(_dot(h, w1r[...]) + b1r[...])
        h = jax.nn.silu(_dot(h, w2r[...]) + b2r[...])
        h = jax.nn.silu(_dot(h, w3r[...]) + b3r[...])
        e = _dot(h, w4r[...]) + b4r[...]
        oh = (owr[...] == lax.broadcasted_iota(jnp.int32, (N, NG), 1))
        oh = oh.astype(jnp.float32)
        dn = (((0,), (0,)), ((), ()))
        esum = lax.dot_general(e, oh, dn,
                               preferred_element_type=jnp.float32)
        cnt = lax.dot_general(jnp.ones_like(e), oh, dn,
                              preferred_element_type=jnp.float32)
        out[...] = esum / jnp.maximum(cnt, 1.0)

    return pl.pallas_call(
        body,
        out_shape=jax.ShapeDtypeStruct((1, NG), jnp.float32),
    )(x, ow2, w1, b1, w2, b2, w3, b3, w4, b4)


def _pad_idx(a):
    a = a.astype(jnp.int32)
    return jnp.pad(a, (0, BP - a.shape[0])).reshape(BP // BLK, BLK)


def kernel(atomic_numbers, atom_graph, directed2undirected, bg_center,
           bg_bond_i, bg_bond_j, atom_owners, bond_bases_ag, bond_bases_bg,
           angle_bases, params):
    p = params
    srcp = _pad_idx(atom_graph[:, 0])
    dstp = _pad_idx(atom_graph[:, 1])
    d2up = _pad_idx(directed2undirected)
    bgcp = _pad_idx(bg_center)
    bgip = _pad_idx(bg_bond_i)
    bgjp = _pad_idx(bg_bond_j)
    abp = jnp.pad(angle_bases, ((0, BP - A), (0, 0)))

    bond, bwag, bwbg = _pre_u(bond_bases_ag, bond_bases_bg,
                              p['bond_emb_w'], p['bw_ag_w'], p['bw_bg_w'])
    angle = (_pre_a(abp[:BH], p['angle_emb_w']),
             _pre_a(abp[BH:], p['angle_emb_w']))
    x = _x0(atomic_numbers.reshape(N, 1).astype(jnp.int32), p['atom_emb'])

    gated3 = _make_gated(('p', 'p', 'p'), E, True, True, False)
    gated4 = _make_gated(('p', 'p', 'h', 'p'), A, True, True, False)
    glin4 = _make_gated(('p', 'p', 'h', 'p'), A, False, False, True)

    def b2(v):
        return v.reshape(1, D)

    bwd, bwg = _gather((bwag, d2up), (bwbg, bgip))
    for i in range(4):
        center, nbr, bd = _gather((x, srcp), (x, dstp), (bond, d2up))
        msg = gated3([center, bd, nbr], bwd,
                     p['ac_cw1'][i], b2(p['ac_cb1'][i]),
                     p['ac_cw2'][i], b2(p['ac_cb2'][i]),
                     p['ac_gw1'][i], b2(p['ac_gb1'][i]),
                     p['ac_gw2'][i], b2(p['ac_gb2'][i]))
        x = _scatter(msg, srcp, x)
        if i < 3:
            ca, bi, bj = _gather((x, bgcp), (bond, bgip), (bond, bgjp))
            bmsg = gated4([bi, bj, angle, ca], bwg,
                          p['bc_cw1'][i], b2(p['bc_cb1'][i]),
                          p['bc_cw2'][i], b2(p['bc_cb2'][i]),
                          p['bc_gw1'][i], b2(p['bc_gb1'][i]),
                          p['bc_gw2'][i], b2(p['bc_gb2'][i]))
            bond = _scatter(bmsg, bgip, bond)
            bi2, bj2 = _gather((bond, bgip), (bond, bgjp))
            angle = glin4([bi2, bj2, angle, ca], None,
                          p['al_cw'][i], b2(p['al_cb'][i]),
                          p['al_gw'][i], b2(p['al_gb'][i]))

    out = _readout(x, atom_owners.reshape(N, 1).astype(jnp.int32),
                   p['h_w1'], b2(p['h_b1']), p['h_w2'], b2(p['h_b2']),
                   p['h_w3'], b2(p['h_b3']), p['h_w4'],
                   p['h_b4'].reshape(1, 1))
    return out.reshape(NG)
